# baseline (device time: 11046 ns/iter reference)
import jax
import jax.numpy as jnp
from jax import lax
from jax.experimental import pallas as pl
from jax.experimental.pallas import tpu as pltpu

M_GLOBAL = 8192
BM = 512


def kernel(x):
    m, n = x.shape
    grid = m // BM

    def body(x_ref, out_ref, acc, recv_buf, send_sem, recv_sem):
        i = pl.program_id(0)
        my_x = lax.axis_index("x")
        my_y = lax.axis_index("y")
        nbr = (1 - my_x, my_y)

        @pl.when(i == 0)
        def _():
            barrier_sem = pltpu.get_barrier_semaphore()
            pl.semaphore_signal(
                barrier_sem, inc=1, device_id=nbr,
                device_id_type=pl.DeviceIdType.MESH,
            )
            pl.semaphore_wait(barrier_sem, 1)

        chunk = jnp.sum(x_ref[:, :], axis=0, keepdims=True)

        @pl.when(i == 0)
        def _():
            acc[:, :] = chunk

        @pl.when(i > 0)
        def _():
            acc[:, :] = acc[:, :] + chunk

        @pl.when(i == grid - 1)
        def _():
            rdma = pltpu.make_async_remote_copy(
                src_ref=acc,
                dst_ref=recv_buf,
                send_sem=send_sem,
                recv_sem=recv_sem,
                device_id=nbr,
                device_id_type=pl.DeviceIdType.MESH,
            )
            rdma.start()
            rdma.wait()
            out_ref[:, :] = (acc[:, :] + recv_buf[:, :]) * (1.0 / M_GLOBAL)

    return pl.pallas_call(
        body,
        grid=(grid,),
        out_shape=jax.ShapeDtypeStruct((1, n), jnp.float32),
        in_specs=[
            pl.BlockSpec((BM, n), lambda i: (i, 0), memory_space=pltpu.VMEM)
        ],
        out_specs=pl.BlockSpec((1, n), lambda i: (0, 0), memory_space=pltpu.VMEM),
        scratch_shapes=[
            pltpu.VMEM((1, n), jnp.float32),
            pltpu.VMEM((1, n), jnp.float32),
            pltpu.SemaphoreType.DMA,
            pltpu.SemaphoreType.DMA,
        ],
        compiler_params=pltpu.CompilerParams(collective_id=0),
    )(x)


# device time: 10891 ns/iter; 1.0142x vs baseline; 1.0142x over previous
import jax
import jax.numpy as jnp
from jax import lax
from jax.experimental import pallas as pl
from jax.experimental.pallas import tpu as pltpu

M_GLOBAL = 8192
BM = 1024


def kernel(x):
    m, n = x.shape
    grid = m // BM

    def body(x_ref, out_ref, acc_a, acc_b, recv_a, recv_b,
             send_sem_a, recv_sem_a, send_sem_b, recv_sem_b):
        i = pl.program_id(0)
        my_x = lax.axis_index("x")
        my_y = lax.axis_index("y")
        nbr = (1 - my_x, my_y)

        def make_rdma(src, dst, ssem, rsem):
            return pltpu.make_async_remote_copy(
                src_ref=src, dst_ref=dst, send_sem=ssem, recv_sem=rsem,
                device_id=nbr, device_id_type=pl.DeviceIdType.MESH,
            )

        @pl.when(i == 0)
        def _():
            barrier_sem = pltpu.get_barrier_semaphore()
            pl.semaphore_signal(
                barrier_sem, inc=1, device_id=nbr,
                device_id_type=pl.DeviceIdType.MESH,
            )

        chunk = jnp.sum(x_ref[:, :], axis=0, keepdims=True)

        @pl.when(i == 0)
        def _():
            acc_a[:, :] = chunk

        @pl.when(i == 1)
        def _():
            acc_a[:, :] = acc_a[:, :] + chunk
            pl.semaphore_wait(pltpu.get_barrier_semaphore(), 1)
            make_rdma(acc_a, recv_a, send_sem_a, recv_sem_a).start()

        @pl.when(i == 2)
        def _():
            acc_b[:, :] = chunk

        @pl.when(i == grid - 1)
        def _():
            acc_b[:, :] = acc_b[:, :] + chunk
            rdma_b = make_rdma(acc_b, recv_b, send_sem_b, recv_sem_b)
            rdma_b.start()
            make_rdma(acc_a, recv_a, send_sem_a, recv_sem_a).wait()
            rdma_b.wait()
            out_ref[:, :] = (
                acc_a[:, :] + acc_b[:, :] + recv_a[:, :] + recv_b[:, :]
            ) * (1.0 / M_GLOBAL)

    return pl.pallas_call(
        body,
        grid=(grid,),
        out_shape=jax.ShapeDtypeStruct((1, n), jnp.float32),
        in_specs=[
            pl.BlockSpec((BM, n), lambda i: (i, 0), memory_space=pltpu.VMEM)
        ],
        out_specs=pl.BlockSpec((1, n), lambda i: (0, 0), memory_space=pltpu.VMEM),
        scratch_shapes=[
            pltpu.VMEM((1, n), jnp.float32),
            pltpu.VMEM((1, n), jnp.float32),
            pltpu.VMEM((1, n), jnp.float32),
            pltpu.VMEM((1, n), jnp.float32),
            pltpu.SemaphoreType.DMA,
            pltpu.SemaphoreType.DMA,
            pltpu.SemaphoreType.DMA,
            pltpu.SemaphoreType.DMA,
        ],
        compiler_params=pltpu.CompilerParams(collective_id=0),
    )(x)


# device time: 10782 ns/iter; 1.0245x vs baseline; 1.0101x over previous
import jax
import jax.numpy as jnp
from jax import lax
from jax.experimental import pallas as pl
from jax.experimental.pallas import tpu as pltpu

M_GLOBAL = 8192


def kernel(x):
    m, n = x.shape

    def body(x_ref, out_ref, send_buf, recv_buf, send_sem, recv_sem):
        my_x = lax.axis_index("x")
        my_y = lax.axis_index("y")
        nbr = (1 - my_x, my_y)

        barrier_sem = pltpu.get_barrier_semaphore()
        pl.semaphore_signal(
            barrier_sem, inc=1, device_id=nbr,
            device_id_type=pl.DeviceIdType.MESH,
        )

        send_buf[:, :] = jnp.sum(x_ref[:, :], axis=0, keepdims=True)

        pl.semaphore_wait(barrier_sem, 1)

        rdma = pltpu.make_async_remote_copy(
            src_ref=send_buf,
            dst_ref=recv_buf,
            send_sem=send_sem,
            recv_sem=recv_sem,
            device_id=nbr,
            device_id_type=pl.DeviceIdType.MESH,
        )
        rdma.start()
        rdma.wait()

        out_ref[:, :] = (send_buf[:, :] + recv_buf[:, :]) * (1.0 / M_GLOBAL)

    return pl.pallas_call(
        body,
        out_shape=jax.ShapeDtypeStruct((1, n), jnp.float32),
        in_specs=[pl.BlockSpec(memory_space=pltpu.VMEM)],
        out_specs=pl.BlockSpec(memory_space=pltpu.VMEM),
        scratch_shapes=[
            pltpu.VMEM((1, n), jnp.float32),
            pltpu.VMEM((1, n), jnp.float32),
            pltpu.SemaphoreType.DMA,
            pltpu.SemaphoreType.DMA,
        ],
        compiler_params=pltpu.CompilerParams(collective_id=0),
    )(x)
